# baseline (speedup 1.0000x reference)
import functools

import jax
import jax.numpy as jnp
from jax.experimental import pallas as pl
from jax.experimental.pallas import tpu as pltpu

_SCALE = 127.0


def _pass_0(x0in_ref, wt_ref, b_ref, xlin_ref, xlint_ref):
    xlin = (
        jnp.dot(x0in_ref[...], wt_ref[...], preferred_element_type=jnp.float32)
        + b_ref[...]
    )
    xlin_ref[...] = xlin
    xlint_ref[...] = xlin.T


def _pass_a(xlint_ref, inc_ref, x1_ref, inc8_ref, ns_ref, *, rem):
    i = pl.program_id(0)

    @pl.when(i == 0)
    def _():
        ns_ref[...] = jnp.zeros_like(ns_ref)

    x1t = jnp.dot(xlint_ref[...], inc_ref[...], preferred_element_type=jnp.float32)
    x1_ref[...] = x1t.T
    inc8_ref[...] = (inc_ref[...] * _SCALE + 0.5).astype(jnp.int8)
    if rem == 0:
        ns_ref[...] += jnp.sum(inc_ref[...], axis=1, keepdims=True)
    else:
        @pl.when(i < pl.num_programs(0) - 1)
        def _():
            ns_ref[...] += jnp.sum(inc_ref[...], axis=1, keepdims=True)

        @pl.when(i == pl.num_programs(0) - 1)
        def _():
            ns_ref[...] += jnp.sum(inc_ref[:, :rem], axis=1, keepdims=True)


def _pass_b(inc8_ref, x1_ref, xlin_ref, ns_ref, out_ref, x1bf_ref):
    @pl.when(pl.program_id(0) == 0)
    def _():
        x1bf_ref[...] = x1_ref[...].astype(jnp.bfloat16)
    acc = jnp.dot(
        inc8_ref[...].astype(jnp.bfloat16), x1bf_ref[...],
        preferred_element_type=jnp.float32,
    )
    out_ref[...] = xlin_ref[...] + acc * (1.0 / _SCALE) / ns_ref[...]


def kernel(x_0, incidence_1, W, b):
    n, c_in = x_0.shape
    e = incidence_1.shape[1]
    c_hid = W.shape[0]
    wt = W.T
    b2 = b.reshape(1, c_hid)

    xlin, xlint = pl.pallas_call(
        _pass_0,
        out_shape=[
            jax.ShapeDtypeStruct((n, c_hid), jnp.float32),
            jax.ShapeDtypeStruct((c_hid, n), jnp.float32),
        ],
    )(x_0, wt, b2)

    be = min(384, e)
    x_1, inc8, ns = pl.pallas_call(
        functools.partial(_pass_a, rem=e % be),
        compiler_params=pltpu.CompilerParams(
            vmem_limit_bytes=63 * 1024 * 1024),
        grid=(pl.cdiv(e, be),),
        in_specs=[
            pl.BlockSpec((c_hid, n), lambda i: (0, 0)),
            pl.BlockSpec((n, be), lambda i: (0, i)),
        ],
        out_specs=[
            pl.BlockSpec((be, c_hid), lambda i: (i, 0)),
            pl.BlockSpec((n, be), lambda i: (0, i)),
            pl.BlockSpec((n, 1), lambda i: (0, 0)),
        ],
        out_shape=[
            jax.ShapeDtypeStruct((e, c_hid), jnp.float32),
            jax.ShapeDtypeStruct((n, e), jnp.int8),
            jax.ShapeDtypeStruct((n, 1), jnp.float32),
        ],
    )(xlint, incidence_1)

    bn = min(512, n)
    x0_out = pl.pallas_call(
        _pass_b,
        grid=(pl.cdiv(n, bn),),
        in_specs=[
            pl.BlockSpec((bn, e), lambda i: (i, 0)),
            pl.BlockSpec((e, c_hid), lambda i: (0, 0)),
            pl.BlockSpec((bn, c_hid), lambda i: (i, 0)),
            pl.BlockSpec((bn, 1), lambda i: (i, 0)),
        ],
        out_specs=pl.BlockSpec((bn, c_hid), lambda i: (i, 0)),
        out_shape=jax.ShapeDtypeStruct((n, c_hid), jnp.float32),
        scratch_shapes=[pltpu.VMEM((e, c_hid), jnp.bfloat16)],
    )(inc8, x_1, xlin, ns)

    return (x0_out, x_1)


# speedup vs baseline: 1.1544x; 1.1544x over previous
import functools

import jax
import jax.numpy as jnp
from jax.experimental import pallas as pl
from jax.experimental.pallas import tpu as pltpu

_SCALE = 127.0


def _pass_a(x0in_ref, inc_ref, wt_ref, b_ref, xlin_ref, x1_ref, inc8_ref, ns_ref,
            *, e_total):
    @pl.when(pl.program_id(0) == 0)
    def _():
        xlin_ref[...] = (
            jnp.dot(x0in_ref[...], wt_ref[...], preferred_element_type=jnp.float32)
            + b_ref[...]
        )
        ns_ref[...] = jnp.zeros_like(ns_ref)
    blk = inc_ref[...]
    x1_ref[...] = jax.lax.dot_general(
        blk, xlin_ref[...],
        dimension_numbers=(((0,), (0,)), ((), ())),
        preferred_element_type=jnp.float32,
    )
    inc8_ref[...] = (blk * _SCALE + 0.5).astype(jnp.int8)
    col = (jax.lax.broadcasted_iota(jnp.int32, blk.shape, 1)
           + pl.program_id(0) * blk.shape[1])
    ns_ref[...] += jnp.sum(jnp.where(col < e_total, blk, 0.0), axis=1,
                           keepdims=True)


def _pass_b(inc8_ref, x1_ref, xlin_ref, ns_ref, out_ref, x1bf_ref):
    @pl.when(pl.program_id(0) == 0)
    def _():
        x1bf_ref[...] = x1_ref[...].astype(jnp.bfloat16)
    acc = jnp.dot(
        inc8_ref[...].astype(jnp.bfloat16), x1bf_ref[...],
        preferred_element_type=jnp.float32,
    )
    out_ref[...] = xlin_ref[...] + acc * (1.0 / _SCALE) / ns_ref[...]


def kernel(x_0, incidence_1, W, b):
    n, c_in = x_0.shape
    e = incidence_1.shape[1]
    c_hid = W.shape[0]
    wt = W.T
    b2 = b.reshape(1, c_hid)

    be = min(384, e)
    xlin, x_1, inc8, ns = pl.pallas_call(
        functools.partial(_pass_a, e_total=e),
        grid=(pl.cdiv(e, be),),
        in_specs=[
            pl.BlockSpec((n, c_in), lambda i: (0, 0)),
            pl.BlockSpec((n, be), lambda i: (0, i)),
            pl.BlockSpec((c_in, c_hid), lambda i: (0, 0)),
            pl.BlockSpec((1, c_hid), lambda i: (0, 0)),
        ],
        out_specs=[
            pl.BlockSpec((n, c_hid), lambda i: (0, 0)),
            pl.BlockSpec((be, c_hid), lambda i: (i, 0)),
            pl.BlockSpec((n, be), lambda i: (0, i)),
            pl.BlockSpec((n, 1), lambda i: (0, 0)),
        ],
        out_shape=[
            jax.ShapeDtypeStruct((n, c_hid), jnp.float32),
            jax.ShapeDtypeStruct((e, c_hid), jnp.float32),
            jax.ShapeDtypeStruct((n, e), jnp.int8),
            jax.ShapeDtypeStruct((n, 1), jnp.float32),
        ],
    )(x_0, incidence_1, wt, b2)

    bn = min(512, n)
    x0_out = pl.pallas_call(
        _pass_b,
        grid=(pl.cdiv(n, bn),),
        in_specs=[
            pl.BlockSpec((bn, e), lambda i: (i, 0)),
            pl.BlockSpec((e, c_hid), lambda i: (0, 0)),
            pl.BlockSpec((bn, c_hid), lambda i: (i, 0)),
            pl.BlockSpec((bn, 1), lambda i: (i, 0)),
        ],
        out_specs=pl.BlockSpec((bn, c_hid), lambda i: (i, 0)),
        out_shape=jax.ShapeDtypeStruct((n, c_hid), jnp.float32),
        scratch_shapes=[pltpu.VMEM((e, c_hid), jnp.bfloat16)],
    )(inc8, x_1, xlin, ns)

    return (x0_out, x_1)


# speedup vs baseline: 1.2763x; 1.1056x over previous
import functools

import jax
import jax.numpy as jnp
from jax.experimental import pallas as pl
from jax.experimental.pallas import tpu as pltpu


def _fused(x0in_ref, inc_ref, wt_ref, b_ref,
           xlin_ref, x1_ref, out_ref,
           xlintbf_ref, macc_ref, nacc_ref, *, e_total):
    i = pl.program_id(0)

    @pl.when(i == 0)
    def _():
        xlin = (
            jnp.dot(x0in_ref[...], wt_ref[...], preferred_element_type=jnp.float32)
            + b_ref[...]
        )
        xlin_ref[...] = xlin
        xlintbf_ref[...] = xlin.T.astype(jnp.bfloat16)
        macc_ref[...] = jnp.zeros_like(macc_ref)
        nacc_ref[...] = jnp.zeros_like(nacc_ref)

    col = (jax.lax.broadcasted_iota(jnp.int32, inc_ref.shape, 1)
           + i * inc_ref.shape[1])
    blkbf = jnp.where(col < e_total, inc_ref[...], 0.0).astype(jnp.bfloat16)

    x1t = jnp.dot(xlintbf_ref[...], blkbf, preferred_element_type=jnp.float32)
    x1 = x1t.T
    x1_ref[...] = x1
    macc_ref[...] += jnp.dot(blkbf, x1.astype(jnp.bfloat16),
                             preferred_element_type=jnp.float32)
    ones = jnp.ones((blkbf.shape[1], nacc_ref.shape[1]), jnp.bfloat16)
    nacc_ref[...] += jnp.dot(blkbf, ones, preferred_element_type=jnp.float32)

    @pl.when(i == pl.num_programs(0) - 1)
    def _():
        out_ref[...] = xlin_ref[...] + macc_ref[...] / nacc_ref[...]


def kernel(x_0, incidence_1, W, b):
    n, c_in = x_0.shape
    e = incidence_1.shape[1]
    c_hid = W.shape[0]
    wt = W.T
    b2 = b.reshape(1, c_hid)

    be = min(256, e)
    xlin, x_1, x0_out = pl.pallas_call(
        functools.partial(_fused, e_total=e),
        grid=(pl.cdiv(e, be),),
        in_specs=[
            pl.BlockSpec((n, c_in), lambda i: (0, 0)),
            pl.BlockSpec((n, be), lambda i: (0, i)),
            pl.BlockSpec((c_in, c_hid), lambda i: (0, 0)),
            pl.BlockSpec((1, c_hid), lambda i: (0, 0)),
        ],
        out_specs=[
            pl.BlockSpec((n, c_hid), lambda i: (0, 0)),
            pl.BlockSpec((be, c_hid), lambda i: (i, 0)),
            pl.BlockSpec((n, c_hid), lambda i: (0, 0)),
        ],
        out_shape=[
            jax.ShapeDtypeStruct((n, c_hid), jnp.float32),
            jax.ShapeDtypeStruct((e, c_hid), jnp.float32),
            jax.ShapeDtypeStruct((n, c_hid), jnp.float32),
        ],
        scratch_shapes=[
            pltpu.VMEM((c_hid, n), jnp.bfloat16),
            pltpu.VMEM((n, c_hid), jnp.float32),
            pltpu.VMEM((n, c_hid), jnp.float32),
        ],
        compiler_params=pltpu.CompilerParams(
            vmem_limit_bytes=63 * 1024 * 1024),
    )(x_0, incidence_1, wt, b2)

    return (x0_out, x_1)


# speedup vs baseline: 1.6071x; 1.2592x over previous
import functools

import jax
import jax.numpy as jnp
from jax.experimental import pallas as pl
from jax.experimental.pallas import tpu as pltpu


def _fused(x0in_ref, inc_ref, wt_ref, b_ref,
           xlin_ref, x1_ref, out_ref,
           xlintbf_ref, macc_ref, nacc_ref, *, e_total):
    i = pl.program_id(0)

    @pl.when(i == 0)
    def _():
        xlin = (
            jnp.dot(x0in_ref[...], wt_ref[...], preferred_element_type=jnp.float32)
            + b_ref[...]
        )
        xlin_ref[...] = xlin
        xlintbf_ref[...] = xlin.T.astype(jnp.bfloat16)
        macc_ref[...] = jnp.zeros_like(macc_ref)
        nacc_ref[...] = jnp.zeros_like(nacc_ref)

    col = (jax.lax.broadcasted_iota(jnp.int32, inc_ref.shape, 1)
           + i * inc_ref.shape[1])
    blkbf = jnp.where(col < e_total, inc_ref[...], 0.0).astype(jnp.bfloat16)

    x1t = jnp.dot(xlintbf_ref[...], blkbf, preferred_element_type=jnp.float32)
    x1 = x1t.T
    x1_ref[...] = x1
    macc_ref[...] += jnp.dot(blkbf, x1.astype(jnp.bfloat16),
                             preferred_element_type=jnp.float32)
    nacc_ref[...] += jnp.sum(blkbf, axis=1, keepdims=True,
                             dtype=jnp.float32)

    @pl.when(i == pl.num_programs(0) - 1)
    def _():
        out_ref[...] = xlin_ref[...] + macc_ref[...] / nacc_ref[...]


def kernel(x_0, incidence_1, W, b):
    n, c_in = x_0.shape
    e = incidence_1.shape[1]
    c_hid = W.shape[0]
    wt = W.T
    b2 = b.reshape(1, c_hid)

    be = min(256, e)
    xlin, x_1, x0_out = pl.pallas_call(
        functools.partial(_fused, e_total=e),
        grid=(pl.cdiv(e, be),),
        in_specs=[
            pl.BlockSpec((n, c_in), lambda i: (0, 0)),
            pl.BlockSpec((n, be), lambda i: (0, i)),
            pl.BlockSpec((c_in, c_hid), lambda i: (0, 0)),
            pl.BlockSpec((1, c_hid), lambda i: (0, 0)),
        ],
        out_specs=[
            pl.BlockSpec((n, c_hid), lambda i: (0, 0)),
            pl.BlockSpec((be, c_hid), lambda i: (i, 0)),
            pl.BlockSpec((n, c_hid), lambda i: (0, 0)),
        ],
        out_shape=[
            jax.ShapeDtypeStruct((n, c_hid), jnp.float32),
            jax.ShapeDtypeStruct((e, c_hid), jnp.float32),
            jax.ShapeDtypeStruct((n, c_hid), jnp.float32),
        ],
        scratch_shapes=[
            pltpu.VMEM((c_hid, n), jnp.bfloat16),
            pltpu.VMEM((n, c_hid), jnp.float32),
            pltpu.VMEM((n, 1), jnp.float32),
        ],
        compiler_params=pltpu.CompilerParams(
            vmem_limit_bytes=63 * 1024 * 1024),
    )(x_0, incidence_1, wt, b2)

    return (x0_out, x_1)


# speedup vs baseline: 1.7260x; 1.0740x over previous
import functools

import jax
import jax.numpy as jnp
from jax.experimental import pallas as pl
from jax.experimental.pallas import tpu as pltpu


def _fused(x0in_ref, inc_ref, wt_ref, b_ref,
           xlin_ref, x1_ref, out_ref,
           xlintbf_ref, nacc_ref, *, e_total):
    i = pl.program_id(0)

    @pl.when(i == 0)
    def _():
        xlin = (
            jnp.dot(x0in_ref[...], wt_ref[...], preferred_element_type=jnp.float32)
            + b_ref[...]
        )
        xlin_ref[...] = xlin
        xlintbf_ref[...] = xlin.T.astype(jnp.bfloat16)
        out_ref[...] = jnp.zeros_like(out_ref)
        nacc_ref[...] = jnp.zeros_like(nacc_ref)

    col = (jax.lax.broadcasted_iota(jnp.int32, inc_ref.shape, 1)
           + i * inc_ref.shape[1])
    blkbf = jnp.where(col < e_total, inc_ref[...], 0.0).astype(jnp.bfloat16)

    x1t = jnp.dot(xlintbf_ref[...], blkbf, preferred_element_type=jnp.float32)
    x1 = x1t.T
    x1_ref[...] = x1
    out_ref[...] += jnp.dot(blkbf, x1.astype(jnp.bfloat16),
                            preferred_element_type=jnp.float32)
    nacc_ref[...] += jnp.sum(blkbf, axis=1, keepdims=True,
                             dtype=jnp.float32)

    @pl.when(i == pl.num_programs(0) - 1)
    def _():
        out_ref[...] = xlin_ref[...] + out_ref[...] / nacc_ref[...]


def kernel(x_0, incidence_1, W, b):
    n, c_in = x_0.shape
    e = incidence_1.shape[1]
    c_hid = W.shape[0]
    wt = W.T
    b2 = b.reshape(1, c_hid)

    be = min(384, e)
    xlin, x_1, x0_out = pl.pallas_call(
        functools.partial(_fused, e_total=e),
        grid=(pl.cdiv(e, be),),
        in_specs=[
            pl.BlockSpec((n, c_in), lambda i: (0, 0)),
            pl.BlockSpec((n, be), lambda i: (0, i)),
            pl.BlockSpec((c_in, c_hid), lambda i: (0, 0)),
            pl.BlockSpec((1, c_hid), lambda i: (0, 0)),
        ],
        out_specs=[
            pl.BlockSpec((n, c_hid), lambda i: (0, 0)),
            pl.BlockSpec((be, c_hid), lambda i: (i, 0)),
            pl.BlockSpec((n, c_hid), lambda i: (0, 0)),
        ],
        out_shape=[
            jax.ShapeDtypeStruct((n, c_hid), jnp.float32),
            jax.ShapeDtypeStruct((e, c_hid), jnp.float32),
            jax.ShapeDtypeStruct((n, c_hid), jnp.float32),
        ],
        scratch_shapes=[
            pltpu.VMEM((c_hid, n), jnp.bfloat16),
            pltpu.VMEM((n, 1), jnp.float32),
        ],
        compiler_params=pltpu.CompilerParams(
            vmem_limit_bytes=63 * 1024 * 1024),
    )(x_0, incidence_1, wt, b2)

    return (x0_out, x_1)
